# Initial kernel scaffold; baseline (speedup 1.0000x reference)
import jax
import jax.numpy as jnp
from jax.experimental import pallas as pl


def kernel(hidden_states, labels, W_sim, b_sim, W_div, b_div, W_meta, b_meta, W_con, b_con, support_features, support_labels):
    raise NotImplementedError("write your pallas kernel here")


# speedup vs baseline: 1.9334x; 1.9334x over previous
import functools

import jax
import jax.numpy as jnp
from jax.experimental import pallas as pl
from jax.experimental.pallas import tpu as pltpu

B, S, HID = 4, 2048, 1024
K = 5
NLAB = 3
NSUP = 100
ROWS = B * S
CHUNK = 512
NSTEP = ROWS // CHUNK
CHUNKS_PER_B = S // CHUNK


def _mm(x, w):
    return jax.lax.dot_general(
        x, w, (((1,), (1,)), ((), ())), preferred_element_type=jnp.float32
    )


def _fsl_kernel(hs_ref, labels_ref, wsim_ref, bsim_ref, wdiv_ref, bdiv_ref,
                wmeta_ref, bmeta_ref, wcon_ref, bcon_ref, supf_ref, supl_ref,
                contrib_ref, sim_ref, div_ref, meta_ref, acc_ref):
    i = pl.program_id(0)

    @pl.when(i == 0)
    def _init():
        acc_ref[...] = jnp.zeros_like(acc_ref)

    b = i // CHUNKS_PER_B
    rowsum = jnp.sum(hs_ref[...], axis=0, keepdims=True)
    acc_ref[pl.ds(b, 1), :] += rowsum

    @pl.when(i == NSTEP - 1)
    def _final():
        pooled = acc_ref[...] * (1.0 / S)

        sim_ref[...] = _mm(pooled, wsim_ref[...]) + bsim_ref[...]
        div_ref[...] = _mm(pooled, wdiv_ref[...]) + bdiv_ref[...]
        tmp = _mm(pooled, wmeta_ref[...]) + bmeta_ref[...]
        meta_ref[...] = _mm(tmp, wcon_ref[...]) + bcon_ref[...]

        e_sel = (jax.lax.broadcasted_iota(jnp.int32, (NSUP, B), 0)
                 == jax.lax.broadcasted_iota(jnp.int32, (NSUP, B), 1)
                 ).astype(jnp.float32)
        pooled_top = jax.lax.dot_general(
            e_sel, pooled, (((1,), (0,)), ((), ())),
            preferred_element_type=jnp.float32)
        row_lt_b = jax.lax.broadcasted_iota(jnp.int32, (NSUP, HID), 0) < B
        supp = jnp.where(row_lt_b, pooled_top, supf_ref[...])
        sims = _mm(pooled, supp)

        labels_f = labels_ref[...].astype(jnp.float32)
        lab_top = jax.lax.dot_general(
            labels_f, e_sel, (((1,), (1,)), ((), ())),
            preferred_element_type=jnp.float32)
        col1 = jax.lax.broadcasted_iota(jnp.int32, (1, NSUP), 1)
        lab_all = jnp.where(col1 < B, lab_top, supl_ref[...].astype(jnp.float32))
        labs_b = jnp.broadcast_to(lab_all, (B, NSUP))

        col = jax.lax.broadcasted_iota(jnp.int32, (B, NSUP), 1)
        vals = sims
        top_v = []
        top_l = []
        for _ in range(K):
            m = jnp.max(vals, axis=1, keepdims=True)
            idx = jnp.min(jnp.where(vals == m, col, NSUP), axis=1,
                          keepdims=True)
            hit = col == idx
            lab_k = jnp.sum(jnp.where(hit, labs_b, 0.0), axis=1,
                            keepdims=True)
            top_v.append(m)
            top_l.append(lab_k)
            vals = jnp.where(hit, -1e30, vals)

        mx = top_v[0]
        es = [jnp.exp(v - mx) for v in top_v]
        den = es[0]
        for e in es[1:]:
            den = den + e
        cls = jax.lax.broadcasted_iota(jnp.int32, (B, NLAB), 1).astype(
            jnp.float32)
        contrib = jnp.zeros((B, NLAB), dtype=jnp.float32)
        for e, lab in zip(es, top_l):
            valid = (lab >= 0.0) & (lab <= NLAB - 1.0)
            onehot = jnp.where((lab == cls) & valid, 1.0, 0.0)
            contrib = contrib + e * onehot
        contrib_ref[...] = contrib / den


@functools.partial(jax.jit, static_argnames=())
def kernel(hidden_states, labels, W_sim, b_sim, W_div, b_div, W_meta, b_meta,
           W_con, b_con, support_features, support_labels):
    hs2 = hidden_states.reshape(ROWS, HID)
    labels2 = labels.reshape(1, B)
    supl2 = support_labels.reshape(1, NSUP)

    full = lambda shape: pl.BlockSpec(shape, lambda i: (0,) * len(shape))
    out_specs = (
        pl.BlockSpec((B, NLAB), lambda i: (0, 0)),
        pl.BlockSpec((B, 128), lambda i: (0, 0)),
        pl.BlockSpec((B, 128), lambda i: (0, 0)),
        pl.BlockSpec((B, 256), lambda i: (0, 0)),
    )
    contrib, sim, div, meta = pl.pallas_call(
        _fsl_kernel,
        grid=(NSTEP,),
        in_specs=[
            pl.BlockSpec((CHUNK, HID), lambda i: (i, 0)),
            full((1, B)),
            full((128, HID)),
            full((1, 128)),
            full((128, HID)),
            full((1, 128)),
            full((HID, HID)),
            full((1, HID)),
            full((256, HID)),
            full((1, 256)),
            full((NSUP, HID)),
            full((1, NSUP)),
        ],
        out_specs=out_specs,
        out_shape=(
            jax.ShapeDtypeStruct((B, NLAB), jnp.float32),
            jax.ShapeDtypeStruct((B, 128), jnp.float32),
            jax.ShapeDtypeStruct((B, 128), jnp.float32),
            jax.ShapeDtypeStruct((B, 256), jnp.float32),
        ),
        scratch_shapes=[pltpu.VMEM((B, HID), jnp.float32)],
    )(hs2, labels2, W_sim, b_sim.reshape(1, 128), W_div, b_div.reshape(1, 128),
      W_meta, b_meta.reshape(1, HID), W_con, b_con.reshape(1, 256),
      support_features, supl2)

    few_shot = jnp.broadcast_to(contrib[:, None, :], (B, S, NLAB))
    return few_shot, sim, div, meta


# speedup vs baseline: 2.3828x; 1.2325x over previous
import functools

import jax
import jax.numpy as jnp
from jax.experimental import pallas as pl
from jax.experimental.pallas import tpu as pltpu

B, S, HID = 4, 2048, 1024
K = 5
NLAB = 3
NSUP = 100
ROWS = B * S
CHUNK = 2048
NSTEP = ROWS // CHUNK
CHUNKS_PER_B = S // CHUNK


def _mm(x, w):
    return jax.lax.dot_general(
        x, w, (((1,), (1,)), ((), ())), preferred_element_type=jnp.float32
    )


def _fsl_kernel(hs_ref, labels_ref, wsim_ref, bsim_ref, wdiv_ref, bdiv_ref,
                wmeta_ref, bmeta_ref, wcon_ref, bcon_ref, supf_ref, supl_ref,
                contrib_ref, sim_ref, div_ref, meta_ref, acc_ref):
    i = pl.program_id(0)

    @pl.when(i == 0)
    def _init():
        acc_ref[...] = jnp.zeros_like(acc_ref)

    b = i // CHUNKS_PER_B
    rowsum = jnp.sum(hs_ref[...], axis=0, keepdims=True)
    acc_ref[pl.ds(b, 1), :] += rowsum

    @pl.when(i == NSTEP - 1)
    def _final():
        pooled = acc_ref[...] * (1.0 / S)

        sim_ref[...] = _mm(pooled, wsim_ref[...]) + bsim_ref[...]
        div_ref[...] = _mm(pooled, wdiv_ref[...]) + bdiv_ref[...]
        tmp = _mm(pooled, wmeta_ref[...]) + bmeta_ref[...]
        meta_ref[...] = _mm(tmp, wcon_ref[...]) + bcon_ref[...]

        e_sel = (jax.lax.broadcasted_iota(jnp.int32, (NSUP, B), 0)
                 == jax.lax.broadcasted_iota(jnp.int32, (NSUP, B), 1)
                 ).astype(jnp.float32)
        pooled_top = jax.lax.dot_general(
            e_sel, pooled, (((1,), (0,)), ((), ())),
            preferred_element_type=jnp.float32)
        row_lt_b = jax.lax.broadcasted_iota(jnp.int32, (NSUP, HID), 0) < B
        supp = jnp.where(row_lt_b, pooled_top, supf_ref[...])
        sims = _mm(pooled, supp)

        labels_f = labels_ref[...].astype(jnp.float32)
        lab_top = jax.lax.dot_general(
            labels_f, e_sel, (((1,), (1,)), ((), ())),
            preferred_element_type=jnp.float32)
        col1 = jax.lax.broadcasted_iota(jnp.int32, (1, NSUP), 1)
        lab_all = jnp.where(col1 < B, lab_top, supl_ref[...].astype(jnp.float32))
        labs_b = jnp.broadcast_to(lab_all, (B, NSUP))

        col = jax.lax.broadcasted_iota(jnp.int32, (B, NSUP), 1)
        vals = sims
        top_v = []
        top_l = []
        for _ in range(K):
            m = jnp.max(vals, axis=1, keepdims=True)
            idx = jnp.min(jnp.where(vals == m, col, NSUP), axis=1,
                          keepdims=True)
            hit = col == idx
            lab_k = jnp.sum(jnp.where(hit, labs_b, 0.0), axis=1,
                            keepdims=True)
            top_v.append(m)
            top_l.append(lab_k)
            vals = jnp.where(hit, -1e30, vals)

        mx = top_v[0]
        es = [jnp.exp(v - mx) for v in top_v]
        den = es[0]
        for e in es[1:]:
            den = den + e
        cls = jax.lax.broadcasted_iota(jnp.int32, (B, NLAB), 1).astype(
            jnp.float32)
        contrib = jnp.zeros((B, NLAB), dtype=jnp.float32)
        for e, lab in zip(es, top_l):
            valid = (lab >= 0.0) & (lab <= NLAB - 1.0)
            onehot = jnp.where((lab == cls) & valid, 1.0, 0.0)
            contrib = contrib + e * onehot
        contrib_ref[...] = contrib / den


@functools.partial(jax.jit, static_argnames=())
def kernel(hidden_states, labels, W_sim, b_sim, W_div, b_div, W_meta, b_meta,
           W_con, b_con, support_features, support_labels):
    hs2 = hidden_states.reshape(ROWS, HID)
    labels2 = labels.reshape(1, B)
    supl2 = support_labels.reshape(1, NSUP)

    full = lambda shape: pl.BlockSpec(shape, lambda i: (0,) * len(shape))
    out_specs = (
        pl.BlockSpec((B, NLAB), lambda i: (0, 0)),
        pl.BlockSpec((B, 128), lambda i: (0, 0)),
        pl.BlockSpec((B, 128), lambda i: (0, 0)),
        pl.BlockSpec((B, 256), lambda i: (0, 0)),
    )
    contrib, sim, div, meta = pl.pallas_call(
        _fsl_kernel,
        grid=(NSTEP,),
        in_specs=[
            pl.BlockSpec((CHUNK, HID), lambda i: (i, 0)),
            full((1, B)),
            full((128, HID)),
            full((1, 128)),
            full((128, HID)),
            full((1, 128)),
            full((HID, HID)),
            full((1, HID)),
            full((256, HID)),
            full((1, 256)),
            full((NSUP, HID)),
            full((1, NSUP)),
        ],
        out_specs=out_specs,
        out_shape=(
            jax.ShapeDtypeStruct((B, NLAB), jnp.float32),
            jax.ShapeDtypeStruct((B, 128), jnp.float32),
            jax.ShapeDtypeStruct((B, 128), jnp.float32),
            jax.ShapeDtypeStruct((B, 256), jnp.float32),
        ),
        scratch_shapes=[pltpu.VMEM((B, HID), jnp.float32)],
    )(hs2, labels2, W_sim, b_sim.reshape(1, 128), W_div, b_div.reshape(1, 128),
      W_meta, b_meta.reshape(1, HID), W_con, b_con.reshape(1, 256),
      support_features, supl2)

    few_shot = jnp.broadcast_to(contrib[:, None, :], (B, S, NLAB))
    return few_shot, sim, div, meta
